# Initial kernel scaffold; baseline (speedup 1.0000x reference)
import jax
import jax.numpy as jnp
from jax.experimental import pallas as pl


def kernel(x_flight, x_airport, ei_fa, ei_af, ei_ff, W_enc_f, b_enc_f, W_enc_a, b_enc_a, basis0, comp0, root0, bias0, gamma0, beta0, basis1, comp1, root1, bias1, gamma1, beta1, W_ro, b_ro):
    raise NotImplementedError("write your pallas kernel here")


# speedup vs baseline: 1.1644x; 1.1644x over previous
import functools

import jax
import jax.numpy as jnp
from jax import lax
from jax.experimental import pallas as pl
from jax.experimental.pallas import tpu as pltpu

N_F = 25000
N_A = 25000
D = 128
E = 200000
NB = 3

ROW_BLOCK = 1000
N_BLOCKS = N_F // ROW_BLOCK


def _enc_body(x_ref, w_ref, b_ref, o_ref):
    o_ref[...] = (
        jnp.dot(x_ref[...], w_ref[...], preferred_element_type=jnp.float32)
        + b_ref[...]
    )


def _encode(x, w, b):
    return pl.pallas_call(
        _enc_body,
        grid=(N_BLOCKS,),
        in_specs=[
            pl.BlockSpec((ROW_BLOCK, D), lambda i: (i, 0)),
            pl.BlockSpec((D, D), lambda i: (0, 0)),
            pl.BlockSpec((1, D), lambda i: (0, 0)),
        ],
        out_specs=pl.BlockSpec((ROW_BLOCK, D), lambda i: (i, 0)),
        out_shape=jax.ShapeDtypeStruct((N_F, D), jnp.float32),
    )(x, w, b.reshape(1, D))


def _rel_weight(comp_arr, basis_ref, r):
    w = comp_arr[r, 0] * basis_ref[0]
    for bb in range(1, NB):
        w = w + comp_arr[r, bb] * basis_ref[bb]
    return w


def _layer_body(h_ref, *refs, rel_ids, readout):
    n_rel = len(rel_ids)
    agg_refs = refs[0:n_rel]
    cnt_refs = refs[n_rel:2 * n_rel]
    root_ref, basis_ref, comp_ref, bias_ref, gamma_ref, beta_ref = refs[
        2 * n_rel:2 * n_rel + 6]
    if readout:
        wro_ref, bro_ref, o_ref = refs[2 * n_rel + 6:]
    else:
        o_ref = refs[2 * n_rel + 6]

    h = h_ref[...]
    acc = jnp.dot(h, root_ref[...], preferred_element_type=jnp.float32)
    acc = acc + bias_ref[...]
    comp_arr = comp_ref[...]
    for (agg_ref, cnt_ref, r) in zip(agg_refs, cnt_refs, rel_ids):
        w = _rel_weight(comp_arr, basis_ref, r)
        mean = agg_ref[...] * (1.0 / jnp.maximum(cnt_ref[...], 1.0))
        acc = acc + jnp.dot(mean, w, preferred_element_type=jnp.float32)

    m = jnp.mean(acc, axis=-1, keepdims=True)
    c = acc - m
    v = jnp.mean(c * c, axis=-1, keepdims=True)
    y = c * lax.rsqrt(v + 1e-5) * gamma_ref[...] + beta_ref[...]
    y = jnp.where(y >= 0.0, y, 0.1 * y)
    y = y + h

    if readout:
        o_ref[...] = (jnp.sum(y * wro_ref[...], axis=-1, keepdims=True)
                      + bro_ref[0, 0])
    else:
        o_ref[...] = y


def _layer(h, aggs, cnts, root, basis, comp, bias, gamma, beta,
           rel_ids, wro=None, bro=None):
    readout = wro is not None
    body = functools.partial(_layer_body, rel_ids=tuple(rel_ids),
                             readout=readout)
    n_rel = len(rel_ids)
    in_specs = [pl.BlockSpec((ROW_BLOCK, D), lambda i: (i, 0))]
    in_specs += [pl.BlockSpec((ROW_BLOCK, D), lambda i: (i, 0))] * n_rel
    in_specs += [pl.BlockSpec((ROW_BLOCK, 1), lambda i: (i, 0))] * n_rel
    in_specs += [
        pl.BlockSpec((D, D), lambda i: (0, 0)),
        pl.BlockSpec((NB, D, D), lambda i: (0, 0, 0)),
        pl.BlockSpec((3, NB), lambda i: (0, 0)),
        pl.BlockSpec((1, D), lambda i: (0, 0)),
        pl.BlockSpec((1, D), lambda i: (0, 0)),
        pl.BlockSpec((1, D), lambda i: (0, 0)),
    ]
    args = [h] + list(aggs) + [c.reshape(-1, 1) for c in cnts] + [
        root, basis, comp, bias.reshape(1, D), gamma.reshape(1, D),
        beta.reshape(1, D)]
    if readout:
        in_specs += [
            pl.BlockSpec((1, D), lambda i: (0, 0)),
            pl.BlockSpec((1, 1), lambda i: (0, 0)),
        ]
        args += [wro.reshape(1, D), bro.reshape(1, 1)]
        out_spec = pl.BlockSpec((ROW_BLOCK, 1), lambda i: (i, 0))
        out_shape = jax.ShapeDtypeStruct((N_F, 1), jnp.float32)
    else:
        out_spec = pl.BlockSpec((ROW_BLOCK, D), lambda i: (i, 0))
        out_shape = jax.ShapeDtypeStruct((N_F, D), jnp.float32)
    return pl.pallas_call(
        body,
        grid=(N_BLOCKS,),
        in_specs=in_specs,
        out_specs=out_spec,
        out_shape=out_shape,
    )(*args)


def _seg_mean_parts(table, src, dst):
    msg = jnp.take(table, src, axis=0)
    agg = jax.ops.segment_sum(msg, dst, num_segments=N_F)
    cnt = jax.ops.segment_sum(jnp.ones((src.shape[0],), jnp.float32), dst,
                              num_segments=N_F)
    return agg, cnt


def kernel(x_flight, x_airport, ei_fa, ei_af, ei_ff, W_enc_f, b_enc_f,
           W_enc_a, b_enc_a, basis0, comp0, root0, bias0, gamma0, beta0,
           basis1, comp1, root1, bias1, gamma1, beta1, W_ro, b_ro):
    src_fa, dst_fa = ei_fa[0], ei_fa[1]
    src_af, dst_af = ei_af[0], ei_af[1]
    src_ff, dst_ff = ei_ff[0], ei_ff[1]

    h_f = _encode(x_flight, W_enc_f, b_enc_f)
    h_a = _encode(x_airport, W_enc_a, b_enc_a)

    agg_fa, cnt_fa = _seg_mean_parts(h_f, src_fa, dst_fa)
    agg_af, cnt_af = _seg_mean_parts(h_a, src_af, dst_af)
    agg_ff, cnt_ff = _seg_mean_parts(h_f, src_ff, dst_ff)

    h_f1 = _layer(h_f, [agg_af, agg_ff], [cnt_af, cnt_ff],
                  root0, basis0, comp0, bias0, gamma0, beta0, rel_ids=(1, 2))
    h_a1 = _layer(h_a, [agg_fa], [cnt_fa],
                  root0, basis0, comp0, bias0, gamma0, beta0, rel_ids=(0,))

    agg_af2, _ = _seg_mean_parts(h_a1, src_af, dst_af)
    agg_ff2, _ = _seg_mean_parts(h_f1, src_ff, dst_ff)

    out = _layer(h_f1, [agg_af2, agg_ff2], [cnt_af, cnt_ff],
                 root1, basis1, comp1, bias1, gamma1, beta1, rel_ids=(1, 2),
                 wro=W_ro, bro=b_ro)
    return out[:, 0]


# speedup vs baseline: 1.4730x; 1.2650x over previous
import functools

import jax
import jax.numpy as jnp
from jax import lax
from jax.experimental import pallas as pl
from jax.experimental.pallas import tpu as pltpu
from jax.experimental.pallas import tpu_sc as plsc

N_F = 25000
N_A = 25000
D = 128
E = 200000
NB = 3

ROW_BLOCK = 1000
N_BLOCKS = N_F // ROW_BLOCK


_SC_NC = 2
_SC_NS = 16
HALF = 12544
TRASH = HALF
ACC_ROWS = HALF + 1
STRIPE = HALF // _SC_NS
CHUNK = 96
N_CHUNKS = 134
PER_TILE = CHUNK * N_CHUNKS
E_PAD = PER_TILE * _SC_NS


def _make_sc_agg(tsel, do_counts):
    n_rel = len(tsel)
    mesh = plsc.VectorSubcoreMesh(core_axis_name="c", subcore_axis_name="s",
                                  num_cores=_SC_NC, num_subcores=_SC_NS)
    n_out = n_rel * 2 if do_counts else n_rel
    out_type = [jax.ShapeDtypeStruct((_SC_NC * HALF, D), jnp.float32)
                for _ in range(n_out)]
    scratch_types = [
        pltpu.VMEM_SHARED((ACC_ROWS, D), jnp.float32),
        pltpu.VMEM((CHUNK, D), jnp.float32),
        pltpu.VMEM((CHUNK, D), jnp.float32),
        pltpu.VMEM((CHUNK,), jnp.int32),
        pltpu.VMEM((CHUNK,), jnp.int32),
        pltpu.VMEM((CHUNK,), jnp.int32),
        pltpu.VMEM((CHUNK,), jnp.int32),
        pltpu.VMEM((CHUNK,), jnp.int32),
        pltpu.VMEM((CHUNK,), jnp.int32),
        pltpu.SemaphoreType.DMA,
        pltpu.SemaphoreType.DMA,
        pltpu.SemaphoreType.DMA,
        pltpu.SemaphoreType.DMA,
    ]

    def body(*refs):
        it = iter(refs)
        h_f = next(it)
        h_a = next(it)
        edges = [(next(it), next(it)) for _ in range(n_rel)]
        outs = [next(it) for _ in range(n_rel)]
        couts = [next(it) for _ in range(n_rel)] if do_counts else None
        acc = next(it)
        rows = (next(it), next(it))
        srcb = (next(it), next(it))
        dstb = (next(it), next(it))
        routed = (next(it), next(it))
        gsem = (next(it), next(it))
        isem = (next(it), next(it))

        c = lax.axis_index("c")
        s = lax.axis_index("s")
        lo = c * HALF
        tables = (h_f, h_a)
        zv = jnp.zeros((16,), jnp.float32)
        base = s * STRIPE
        ebase = s * PER_TILE

        def fill_rows(buf, val):
            vvec = jnp.full((16,), val, jnp.float32)

            def _frow(i, _):
                def _fcol(j, _2):
                    buf[i, pl.ds(j * 16, 16)] = vvec
                    return 0
                lax.fori_loop(0, D // 16, _fcol, 0)
                return 0
            lax.fori_loop(0, CHUNK, _frow, 0)

        def zero_stripe():
            for t in range(8):
                pltpu.sync_copy(rows[1],
                                acc.at[pl.ds(base + t * CHUNK, CHUNK)])
            pltpu.sync_copy(rows[1].at[pl.ds(0, 16)],
                            acc.at[pl.ds(base + 8 * CHUNK, 16)])

        def copy_out(dst_hbm):
            obase = c * HALF + base
            for t in range(8):
                pltpu.sync_copy(acc.at[pl.ds(base + t * CHUNK, CHUNK)],
                                rows[1])
                pltpu.sync_copy(rows[1],
                                dst_hbm.at[pl.ds(obase + t * CHUNK, CHUNK)])
            pltpu.sync_copy(acc.at[pl.ds(base + 8 * CHUNK, 16)],
                            rows[1].at[pl.ds(0, 16)])
            pltpu.sync_copy(rows[1].at[pl.ds(0, 16)],
                            dst_hbm.at[pl.ds(obase + 8 * CHUNK, 16)])

        def route(db, rb):
            def inner(j, _):
                d = db[pl.ds(j * 16, 16)]
                loc = d - lo
                ok = jnp.logical_and(loc >= 0, loc < HALF)
                rb[pl.ds(j * 16, 16)] = jnp.where(ok, loc, TRASH)
                return 0
            lax.fori_loop(0, CHUNK // 16, inner, 0)

        def rows_phase(table, src_hbm, dst_hbm, out_hbm):
            fill_rows(rows[1], 0.0)
            zero_stripe()
            plsc.subcore_barrier()

            def idx_load(i, b, sync=False):
                sslice = src_hbm.at[pl.ds(ebase + i * CHUNK, CHUNK)]
                dslice = dst_hbm.at[pl.ds(ebase + i * CHUNK, CHUNK)]
                if sync:
                    pltpu.sync_copy(sslice, srcb[b])
                    pltpu.sync_copy(dslice, dstb[b])
                else:
                    pltpu.async_copy(sslice, srcb[b], isem[b])
                    pltpu.async_copy(dslice, dstb[b], isem[b])

            def idx_wait(i, b):
                pltpu.make_async_copy(
                    src_hbm.at[pl.ds(ebase + i * CHUNK, CHUNK)],
                    srcb[b], isem[b]).wait()
                pltpu.make_async_copy(
                    dst_hbm.at[pl.ds(ebase + i * CHUNK, CHUNK)],
                    dstb[b], isem[b]).wait()

            def gather(i, b):
                pltpu.async_copy(table.at[srcb[b]], rows[b], gsem[b])

            def gather_wait(i, b):
                pltpu.make_async_copy(table.at[srcb[b]], rows[b],
                                      gsem[b]).wait()

            idx_load(0, 0, sync=True)
            gather(0, 0)
            idx_load(1, 1)

            def phase(i, b):
                route(dstb[b], routed[b])
                gather_wait(i, b)
                pltpu.sync_copy(rows[b], acc.at[routed[b]], add=True)

                @pl.when(i + 1 < N_CHUNKS)
                def _issue_next():
                    idx_wait(i + 1, 1 - b)
                    gather(i + 1, 1 - b)

                @pl.when(i + 2 < N_CHUNKS)
                def _prefetch_idx():
                    idx_load(i + 2, b)

            def loop_body(k, _):
                phase(2 * k, 0)
                phase(2 * k + 1, 1)
                return 0
            lax.fori_loop(0, N_CHUNKS // 2, loop_body, 0)
            plsc.subcore_barrier()
            copy_out(out_hbm)

        def count_phase(dst_hbm, out_hbm):
            fill_rows(rows[1], 0.0)
            zero_stripe()
            plsc.subcore_barrier()
            fill_rows(rows[0], 1.0)

            def dst_load(i, b, sync=False):
                dslice = dst_hbm.at[pl.ds(ebase + i * CHUNK, CHUNK)]
                if sync:
                    pltpu.sync_copy(dslice, dstb[b])
                else:
                    pltpu.async_copy(dslice, dstb[b], isem[b])

            def dst_wait(i, b):
                pltpu.make_async_copy(
                    dst_hbm.at[pl.ds(ebase + i * CHUNK, CHUNK)],
                    dstb[b], isem[b]).wait()

            dst_load(0, 0, sync=True)
            dst_load(1, 1)

            def phase(i, b):
                @pl.when(jnp.logical_and(i >= 1, i < N_CHUNKS))
                def _wait_idx():
                    dst_wait(i, b)
                route(dstb[b], routed[b])
                pltpu.sync_copy(rows[0], acc.at[routed[b]], add=True)

                @pl.when(i + 2 < N_CHUNKS)
                def _prefetch_idx():
                    dst_load(i + 2, b)

            def loop_body(k, _):
                phase(2 * k, 0)
                phase(2 * k + 1, 1)
                return 0
            lax.fori_loop(0, N_CHUNKS // 2, loop_body, 0)
            plsc.subcore_barrier()
            copy_out(out_hbm)

        for r in range(n_rel):
            rows_phase(tables[tsel[r]], edges[r][0], edges[r][1], outs[r])
        if do_counts:
            for r in range(n_rel):
                count_phase(edges[r][1], couts[r])

    return pl.kernel(
        body, out_type=out_type, mesh=mesh, scratch_types=scratch_types,
        compiler_params=pltpu.CompilerParams(needs_layout_passes=False))


_sc_agg_l0 = _make_sc_agg((0, 1, 0), do_counts=True)
_sc_agg_l1 = _make_sc_agg((1, 0), do_counts=False)


def _prep_edges(ei):
    src = ei[0].astype(jnp.int32)
    dst = ei[1].astype(jnp.int32)
    src = jnp.pad(src, (0, E_PAD - E))
    dst = jnp.pad(dst, (0, E_PAD - E), constant_values=jnp.int32(1 << 30))
    return src, dst


def _enc_body(x_ref, w_ref, b_ref, o_ref):
    o_ref[...] = (
        jnp.dot(x_ref[...], w_ref[...], preferred_element_type=jnp.float32)
        + b_ref[...]
    )


def _encode(x, w, b):
    return pl.pallas_call(
        _enc_body,
        grid=(N_BLOCKS,),
        in_specs=[
            pl.BlockSpec((ROW_BLOCK, D), lambda i: (i, 0)),
            pl.BlockSpec((D, D), lambda i: (0, 0)),
            pl.BlockSpec((1, D), lambda i: (0, 0)),
        ],
        out_specs=pl.BlockSpec((ROW_BLOCK, D), lambda i: (i, 0)),
        out_shape=jax.ShapeDtypeStruct((N_F, D), jnp.float32),
    )(x, w, b.reshape(1, D))


def _rel_weight(comp_arr, basis_ref, r):
    w = comp_arr[r, 0] * basis_ref[0]
    for bb in range(1, NB):
        w = w + comp_arr[r, bb] * basis_ref[bb]
    return w


def _layer_body(h_ref, *refs, rel_ids, readout):
    n_rel = len(rel_ids)
    agg_refs = refs[0:n_rel]
    cnt_refs = refs[n_rel:2 * n_rel]
    root_ref, basis_ref, comp_ref, bias_ref, gamma_ref, beta_ref = refs[
        2 * n_rel:2 * n_rel + 6]
    if readout:
        wro_ref, bro_ref, o_ref = refs[2 * n_rel + 6:]
    else:
        o_ref = refs[2 * n_rel + 6]

    h = h_ref[...]
    acc = jnp.dot(h, root_ref[...], preferred_element_type=jnp.float32)
    acc = acc + bias_ref[...]
    comp_arr = comp_ref[...]
    for (agg_ref, cnt_ref, r) in zip(agg_refs, cnt_refs, rel_ids):
        w = _rel_weight(comp_arr, basis_ref, r)
        mean = agg_ref[...] * (1.0 / jnp.maximum(cnt_ref[...], 1.0))
        acc = acc + jnp.dot(mean, w, preferred_element_type=jnp.float32)

    m = jnp.mean(acc, axis=-1, keepdims=True)
    c = acc - m
    v = jnp.mean(c * c, axis=-1, keepdims=True)
    y = c * lax.rsqrt(v + 1e-5) * gamma_ref[...] + beta_ref[...]
    y = jnp.where(y >= 0.0, y, 0.1 * y)
    y = y + h

    if readout:
        o_ref[...] = (jnp.sum(y * wro_ref[...], axis=-1, keepdims=True)
                      + bro_ref[0, 0])
    else:
        o_ref[...] = y


def _layer(h, aggs, cnts, root, basis, comp, bias, gamma, beta,
           rel_ids, wro=None, bro=None):
    readout = wro is not None
    body = functools.partial(_layer_body, rel_ids=tuple(rel_ids),
                             readout=readout)
    n_rel = len(rel_ids)
    in_specs = [pl.BlockSpec((ROW_BLOCK, D), lambda i: (i, 0))]
    in_specs += [pl.BlockSpec((ROW_BLOCK, D), lambda i: (i, 0))] * n_rel
    in_specs += [pl.BlockSpec((ROW_BLOCK, 1), lambda i: (i, 0))] * n_rel
    in_specs += [
        pl.BlockSpec((D, D), lambda i: (0, 0)),
        pl.BlockSpec((NB, D, D), lambda i: (0, 0, 0)),
        pl.BlockSpec((3, NB), lambda i: (0, 0)),
        pl.BlockSpec((1, D), lambda i: (0, 0)),
        pl.BlockSpec((1, D), lambda i: (0, 0)),
        pl.BlockSpec((1, D), lambda i: (0, 0)),
    ]
    args = [h] + list(aggs) + [c.reshape(-1, 1) for c in cnts] + [
        root, basis, comp, bias.reshape(1, D), gamma.reshape(1, D),
        beta.reshape(1, D)]
    if readout:
        in_specs += [
            pl.BlockSpec((1, D), lambda i: (0, 0)),
            pl.BlockSpec((1, 1), lambda i: (0, 0)),
        ]
        args += [wro.reshape(1, D), bro.reshape(1, 1)]
        out_spec = pl.BlockSpec((ROW_BLOCK, 1), lambda i: (i, 0))
        out_shape = jax.ShapeDtypeStruct((N_F, 1), jnp.float32)
    else:
        out_spec = pl.BlockSpec((ROW_BLOCK, D), lambda i: (i, 0))
        out_shape = jax.ShapeDtypeStruct((N_F, D), jnp.float32)
    return pl.pallas_call(
        body,
        grid=(N_BLOCKS,),
        in_specs=in_specs,
        out_specs=out_spec,
        out_shape=out_shape,
    )(*args)


def kernel(x_flight, x_airport, ei_fa, ei_af, ei_ff, W_enc_f, b_enc_f,
           W_enc_a, b_enc_a, basis0, comp0, root0, bias0, gamma0, beta0,
           basis1, comp1, root1, bias1, gamma1, beta1, W_ro, b_ro):
    src_fa, dst_fa = _prep_edges(ei_fa)
    src_af, dst_af = _prep_edges(ei_af)
    src_ff, dst_ff = _prep_edges(ei_ff)

    h_f = _encode(x_flight, W_enc_f, b_enc_f)
    h_a = _encode(x_airport, W_enc_a, b_enc_a)

    l0 = _sc_agg_l0(h_f, h_a, src_fa, dst_fa, src_af, dst_af, src_ff, dst_ff)
    agg_fa, agg_af, agg_ff = (o[:N_F] for o in l0[:3])
    cnt_fa, cnt_af, cnt_ff = (o[:N_F, 0] for o in l0[3:])

    h_f1 = _layer(h_f, [agg_af, agg_ff], [cnt_af, cnt_ff],
                  root0, basis0, comp0, bias0, gamma0, beta0, rel_ids=(1, 2))
    h_a1 = _layer(h_a, [agg_fa], [cnt_fa],
                  root0, basis0, comp0, bias0, gamma0, beta0, rel_ids=(0,))

    l1 = _sc_agg_l1(h_f1, h_a1, src_af, dst_af, src_ff, dst_ff)
    agg_af2, agg_ff2 = (o[:N_F] for o in l1)

    out = _layer(h_f1, [agg_af2, agg_ff2], [cnt_af, cnt_ff],
                 root1, basis1, comp1, bias1, gamma1, beta1, rel_ids=(1, 2),
                 wro=W_ro, bro=b_ro)
    return out[:, 0]


# speedup vs baseline: 1.6214x; 1.1007x over previous
import functools

import jax
import jax.numpy as jnp
from jax import lax
from jax.experimental import pallas as pl
from jax.experimental.pallas import tpu as pltpu
from jax.experimental.pallas import tpu_sc as plsc

N_F = 25000
N_A = 25000
D = 128
E = 200000
NB = 3

ROW_BLOCK = 1000
N_BLOCKS = N_F // ROW_BLOCK


_SC_NC = 2
_SC_NS = 16
HALF = 12544
TRASH = HALF
ACC_ROWS = HALF + 16
STRIPE = HALF // _SC_NS
CHUNK = 96
N_CHUNKS = 134
PER_TILE = CHUNK * N_CHUNKS
E_PAD = PER_TILE * _SC_NS


def _make_sc_agg(tsel, do_counts):
    n_rel = len(tsel)
    mesh = plsc.VectorSubcoreMesh(core_axis_name="c", subcore_axis_name="s",
                                  num_cores=_SC_NC, num_subcores=_SC_NS)
    n_out = n_rel * 2 if do_counts else n_rel
    out_type = [jax.ShapeDtypeStruct((_SC_NC * HALF, D), jnp.float32)
                for _ in range(n_out)]
    scratch_types = [
        pltpu.VMEM_SHARED((ACC_ROWS, D), jnp.float32),
        pltpu.VMEM((CHUNK, D), jnp.float32),
        pltpu.VMEM((CHUNK, D), jnp.float32),
        pltpu.VMEM((CHUNK,), jnp.int32),
        pltpu.VMEM((CHUNK,), jnp.int32),
        pltpu.VMEM((CHUNK,), jnp.int32),
        pltpu.VMEM((CHUNK,), jnp.int32),
        pltpu.VMEM((CHUNK,), jnp.int32),
        pltpu.VMEM((CHUNK,), jnp.int32),
        pltpu.SemaphoreType.DMA,
        pltpu.SemaphoreType.DMA,
        pltpu.SemaphoreType.DMA,
        pltpu.SemaphoreType.DMA,
    ]

    def body(*refs):
        it = iter(refs)
        h_f = next(it)
        h_a = next(it)
        edges = [(next(it), next(it)) for _ in range(n_rel)]
        outs = [next(it) for _ in range(n_rel)]
        couts = [next(it) for _ in range(n_rel)] if do_counts else None
        acc = next(it)
        rows = (next(it), next(it))
        srcb = (next(it), next(it))
        dstb = (next(it), next(it))
        routed = (next(it), next(it))
        gsem = (next(it), next(it))
        isem = (next(it), next(it))

        c = lax.axis_index("c")
        s = lax.axis_index("s")
        lo = c * HALF
        tables = (h_f, h_a)
        zv = jnp.zeros((16,), jnp.float32)
        base = s * STRIPE
        ebase = s * PER_TILE

        def fill_rows(buf, val):
            vvec = jnp.full((16,), val, jnp.float32)

            def _frow(i, _):
                def _fcol(j, _2):
                    buf[i, pl.ds(j * 16, 16)] = vvec
                    return 0
                lax.fori_loop(0, D // 16, _fcol, 0)
                return 0
            lax.fori_loop(0, CHUNK, _frow, 0)

        def zero_stripe():
            for t in range(8):
                pltpu.sync_copy(rows[1],
                                acc.at[pl.ds(base + t * CHUNK, CHUNK)])
            pltpu.sync_copy(rows[1].at[pl.ds(0, 16)],
                            acc.at[pl.ds(base + 8 * CHUNK, 16)])

        def copy_out(dst_hbm):
            obase = c * HALF + base
            for t in range(8):
                pltpu.sync_copy(acc.at[pl.ds(base + t * CHUNK, CHUNK)],
                                rows[1])
                pltpu.sync_copy(rows[1],
                                dst_hbm.at[pl.ds(obase + t * CHUNK, CHUNK)])
            pltpu.sync_copy(acc.at[pl.ds(base + 8 * CHUNK, 16)],
                            rows[1].at[pl.ds(0, 16)])
            pltpu.sync_copy(rows[1].at[pl.ds(0, 16)],
                            dst_hbm.at[pl.ds(obase + 8 * CHUNK, 16)])

        trash_rows = TRASH + lax.iota(jnp.int32, 16)

        def route(db, rb):
            def inner(j, _):
                d = db[pl.ds(j * 16, 16)]
                loc = d - lo
                ok = jnp.logical_and(loc >= 0, loc < HALF)
                rb[pl.ds(j * 16, 16)] = jnp.where(ok, loc, trash_rows)
                return 0
            lax.fori_loop(0, CHUNK // 16, inner, 0)

        def rows_phase(table, src_hbm, dst_hbm, out_hbm):
            fill_rows(rows[1], 0.0)
            zero_stripe()
            plsc.subcore_barrier()

            def idx_load(i, b, sync=False):
                sslice = src_hbm.at[pl.ds(ebase + i * CHUNK, CHUNK)]
                dslice = dst_hbm.at[pl.ds(ebase + i * CHUNK, CHUNK)]
                if sync:
                    pltpu.sync_copy(sslice, srcb[b])
                    pltpu.sync_copy(dslice, dstb[b])
                else:
                    pltpu.async_copy(sslice, srcb[b], isem[b])
                    pltpu.async_copy(dslice, dstb[b], isem[b])

            def idx_wait(i, b):
                pltpu.make_async_copy(
                    src_hbm.at[pl.ds(ebase + i * CHUNK, CHUNK)],
                    srcb[b], isem[b]).wait()
                pltpu.make_async_copy(
                    dst_hbm.at[pl.ds(ebase + i * CHUNK, CHUNK)],
                    dstb[b], isem[b]).wait()

            def gather(i, b):
                pltpu.async_copy(table.at[srcb[b]], rows[b], gsem[b])

            def gather_wait(i, b):
                pltpu.make_async_copy(table.at[srcb[b]], rows[b],
                                      gsem[b]).wait()

            idx_load(0, 0, sync=True)
            gather(0, 0)
            idx_load(1, 1)

            def phase(i, b):
                route(dstb[b], routed[b])
                gather_wait(i, b)

                @pl.when(i + 1 < N_CHUNKS)
                def _issue_next():
                    idx_wait(i + 1, 1 - b)
                    gather(i + 1, 1 - b)

                pltpu.sync_copy(rows[b], acc.at[routed[b]], add=True)

                @pl.when(i + 2 < N_CHUNKS)
                def _prefetch_idx():
                    idx_load(i + 2, b)

            def loop_body(k, _):
                phase(2 * k, 0)
                phase(2 * k + 1, 1)
                return 0
            lax.fori_loop(0, N_CHUNKS // 2, loop_body, 0)
            plsc.subcore_barrier()
            copy_out(out_hbm)

        def count_phase(dst_hbm, out_hbm):
            fill_rows(rows[1], 0.0)
            zero_stripe()
            plsc.subcore_barrier()
            fill_rows(rows[0], 1.0)

            def dst_load(i, b, sync=False):
                dslice = dst_hbm.at[pl.ds(ebase + i * CHUNK, CHUNK)]
                if sync:
                    pltpu.sync_copy(dslice, dstb[b])
                else:
                    pltpu.async_copy(dslice, dstb[b], isem[b])

            def dst_wait(i, b):
                pltpu.make_async_copy(
                    dst_hbm.at[pl.ds(ebase + i * CHUNK, CHUNK)],
                    dstb[b], isem[b]).wait()

            dst_load(0, 0, sync=True)
            dst_load(1, 1)

            def phase(i, b):
                @pl.when(jnp.logical_and(i >= 1, i < N_CHUNKS))
                def _wait_idx():
                    dst_wait(i, b)
                route(dstb[b], routed[b])
                pltpu.sync_copy(rows[0], acc.at[routed[b]], add=True)

                @pl.when(i + 2 < N_CHUNKS)
                def _prefetch_idx():
                    dst_load(i + 2, b)

            def loop_body(k, _):
                phase(2 * k, 0)
                phase(2 * k + 1, 1)
                return 0
            lax.fori_loop(0, N_CHUNKS // 2, loop_body, 0)
            plsc.subcore_barrier()
            copy_out(out_hbm)

        for r in range(n_rel):
            rows_phase(tables[tsel[r]], edges[r][0], edges[r][1], outs[r])
        if do_counts:
            for r in range(n_rel):
                count_phase(edges[r][1], couts[r])

    return pl.kernel(
        body, out_type=out_type, mesh=mesh, scratch_types=scratch_types,
        compiler_params=pltpu.CompilerParams(needs_layout_passes=False))


_sc_agg_l0 = _make_sc_agg((0, 1, 0), do_counts=True)
_sc_agg_l1 = _make_sc_agg((1, 0), do_counts=False)


def _prep_edges(ei):
    src = ei[0].astype(jnp.int32)
    dst = ei[1].astype(jnp.int32)
    src = jnp.pad(src, (0, E_PAD - E))
    dst = jnp.pad(dst, (0, E_PAD - E), constant_values=jnp.int32(1 << 30))
    return src, dst


def _enc_body(x_ref, w_ref, b_ref, o_ref):
    o_ref[...] = (
        jnp.dot(x_ref[...], w_ref[...], preferred_element_type=jnp.float32)
        + b_ref[...]
    )


def _encode(x, w, b):
    return pl.pallas_call(
        _enc_body,
        grid=(N_BLOCKS,),
        in_specs=[
            pl.BlockSpec((ROW_BLOCK, D), lambda i: (i, 0)),
            pl.BlockSpec((D, D), lambda i: (0, 0)),
            pl.BlockSpec((1, D), lambda i: (0, 0)),
        ],
        out_specs=pl.BlockSpec((ROW_BLOCK, D), lambda i: (i, 0)),
        out_shape=jax.ShapeDtypeStruct((N_F, D), jnp.float32),
    )(x, w, b.reshape(1, D))


def _rel_weight(comp_arr, basis_ref, r):
    w = comp_arr[r, 0] * basis_ref[0]
    for bb in range(1, NB):
        w = w + comp_arr[r, bb] * basis_ref[bb]
    return w


def _layer_body(h_ref, *refs, rel_ids, readout):
    n_rel = len(rel_ids)
    agg_refs = refs[0:n_rel]
    cnt_refs = refs[n_rel:2 * n_rel]
    root_ref, basis_ref, comp_ref, bias_ref, gamma_ref, beta_ref = refs[
        2 * n_rel:2 * n_rel + 6]
    if readout:
        wro_ref, bro_ref, o_ref = refs[2 * n_rel + 6:]
    else:
        o_ref = refs[2 * n_rel + 6]

    h = h_ref[...]
    acc = jnp.dot(h, root_ref[...], preferred_element_type=jnp.float32)
    acc = acc + bias_ref[...]
    comp_arr = comp_ref[...]
    for (agg_ref, cnt_ref, r) in zip(agg_refs, cnt_refs, rel_ids):
        w = _rel_weight(comp_arr, basis_ref, r)
        mean = agg_ref[...] * (1.0 / jnp.maximum(cnt_ref[...], 1.0))
        acc = acc + jnp.dot(mean, w, preferred_element_type=jnp.float32)

    m = jnp.mean(acc, axis=-1, keepdims=True)
    c = acc - m
    v = jnp.mean(c * c, axis=-1, keepdims=True)
    y = c * lax.rsqrt(v + 1e-5) * gamma_ref[...] + beta_ref[...]
    y = jnp.where(y >= 0.0, y, 0.1 * y)
    y = y + h

    if readout:
        o_ref[...] = (jnp.sum(y * wro_ref[...], axis=-1, keepdims=True)
                      + bro_ref[0, 0])
    else:
        o_ref[...] = y


def _layer(h, aggs, cnts, root, basis, comp, bias, gamma, beta,
           rel_ids, wro=None, bro=None):
    readout = wro is not None
    body = functools.partial(_layer_body, rel_ids=tuple(rel_ids),
                             readout=readout)
    n_rel = len(rel_ids)
    in_specs = [pl.BlockSpec((ROW_BLOCK, D), lambda i: (i, 0))]
    in_specs += [pl.BlockSpec((ROW_BLOCK, D), lambda i: (i, 0))] * n_rel
    in_specs += [pl.BlockSpec((ROW_BLOCK, 1), lambda i: (i, 0))] * n_rel
    in_specs += [
        pl.BlockSpec((D, D), lambda i: (0, 0)),
        pl.BlockSpec((NB, D, D), lambda i: (0, 0, 0)),
        pl.BlockSpec((3, NB), lambda i: (0, 0)),
        pl.BlockSpec((1, D), lambda i: (0, 0)),
        pl.BlockSpec((1, D), lambda i: (0, 0)),
        pl.BlockSpec((1, D), lambda i: (0, 0)),
    ]
    args = [h] + list(aggs) + [c.reshape(-1, 1) for c in cnts] + [
        root, basis, comp, bias.reshape(1, D), gamma.reshape(1, D),
        beta.reshape(1, D)]
    if readout:
        in_specs += [
            pl.BlockSpec((1, D), lambda i: (0, 0)),
            pl.BlockSpec((1, 1), lambda i: (0, 0)),
        ]
        args += [wro.reshape(1, D), bro.reshape(1, 1)]
        out_spec = pl.BlockSpec((ROW_BLOCK, 1), lambda i: (i, 0))
        out_shape = jax.ShapeDtypeStruct((N_F, 1), jnp.float32)
    else:
        out_spec = pl.BlockSpec((ROW_BLOCK, D), lambda i: (i, 0))
        out_shape = jax.ShapeDtypeStruct((N_F, D), jnp.float32)
    return pl.pallas_call(
        body,
        grid=(N_BLOCKS,),
        in_specs=in_specs,
        out_specs=out_spec,
        out_shape=out_shape,
    )(*args)


def kernel(x_flight, x_airport, ei_fa, ei_af, ei_ff, W_enc_f, b_enc_f,
           W_enc_a, b_enc_a, basis0, comp0, root0, bias0, gamma0, beta0,
           basis1, comp1, root1, bias1, gamma1, beta1, W_ro, b_ro):
    src_fa, dst_fa = _prep_edges(ei_fa)
    src_af, dst_af = _prep_edges(ei_af)
    src_ff, dst_ff = _prep_edges(ei_ff)

    h_f = _encode(x_flight, W_enc_f, b_enc_f)
    h_a = _encode(x_airport, W_enc_a, b_enc_a)

    l0 = _sc_agg_l0(h_f, h_a, src_fa, dst_fa, src_af, dst_af, src_ff, dst_ff)
    agg_fa, agg_af, agg_ff = (o[:N_F] for o in l0[:3])
    cnt_fa, cnt_af, cnt_ff = (o[:N_F, 0] for o in l0[3:])

    h_f1 = _layer(h_f, [agg_af, agg_ff], [cnt_af, cnt_ff],
                  root0, basis0, comp0, bias0, gamma0, beta0, rel_ids=(1, 2))
    h_a1 = _layer(h_a, [agg_fa], [cnt_fa],
                  root0, basis0, comp0, bias0, gamma0, beta0, rel_ids=(0,))

    l1 = _sc_agg_l1(h_f1, h_a1, src_af, dst_af, src_ff, dst_ff)
    agg_af2, agg_ff2 = (o[:N_F] for o in l1)

    out = _layer(h_f1, [agg_af2, agg_ff2], [cnt_af, cnt_ff],
                 root1, basis1, comp1, bias1, gamma1, beta1, rel_ids=(1, 2),
                 wro=W_ro, bro=b_ro)
    return out[:, 0]
